# Initial kernel scaffold; baseline (speedup 1.0000x reference)
import jax
import jax.numpy as jnp
from jax.experimental import pallas as pl


def kernel(x, W):
    raise NotImplementedError("write your pallas kernel here")


# speedup vs baseline: 1.0458x; 1.0458x over previous
import functools

import jax
import jax.numpy as jnp
from jax.experimental import pallas as pl

_EMBED = 768
_NEXP = 8
_LANES = 128
_BLK = 2048


def _router_body(x_ref, wt_ref, g_ref, i_ref, tl_ref, load_ref):
    logits = jnp.dot(x_ref[:], wt_ref[:], preferred_element_type=jnp.float32)
    eidx = jax.lax.broadcasted_iota(jnp.int32, logits.shape, 1)
    neg = jnp.float32(-jnp.inf)
    masked = jnp.where(eidx < _NEXP, logits, neg)

    l1 = jnp.max(masked, axis=1, keepdims=True)
    i1 = jnp.min(jnp.where(masked == l1, eidx, _LANES), axis=1, keepdims=True)
    masked2 = jnp.where(eidx == i1, neg, masked)
    l2 = jnp.max(masked2, axis=1, keepdims=True)
    i2 = jnp.min(jnp.where(masked2 == l2, eidx, _LANES), axis=1, keepdims=True)

    e21 = jnp.exp(l2 - l1)
    denom = 1.0 + e21
    g1 = 1.0 / denom
    g2 = e21 / denom

    g_ref[:, 0:1] = g1
    g_ref[:, 1:2] = g2
    i_ref[:, 0:1] = i1
    i_ref[:, 1:2] = i2
    tl_ref[:, 0:1] = l1
    tl_ref[:, 1:2] = l2

    part = jnp.sum(
        jnp.where(eidx == i1, g1, 0.0) + jnp.where(eidx == i2, g2, 0.0),
        axis=0,
        keepdims=True,
    )

    @pl.when(pl.program_id(0) == 0)
    def _init():
        load_ref[:] = jnp.zeros_like(load_ref)

    load_ref[:] += part


@functools.partial(jax.jit, static_argnames=("interpret",))
def kernel(x, W, interpret=False):
    b, s, d = x.shape
    n = b * s
    x2 = x.reshape(n, d)
    wt = jnp.zeros((d, _LANES), jnp.float32).at[:, :_NEXP].set(W.T)

    grid = (n // _BLK,)
    gates2, idx2, tl2, load = pl.pallas_call(
        _router_body,
        grid=grid,
        in_specs=[
            pl.BlockSpec((_BLK, d), lambda i: (i, 0)),
            pl.BlockSpec((d, _LANES), lambda i: (0, 0)),
        ],
        out_specs=[
            pl.BlockSpec((_BLK, 2), lambda i: (i, 0)),
            pl.BlockSpec((_BLK, 2), lambda i: (i, 0)),
            pl.BlockSpec((_BLK, 2), lambda i: (i, 0)),
            pl.BlockSpec((1, _LANES), lambda i: (0, 0)),
        ],
        out_shape=[
            jax.ShapeDtypeStruct((n, 2), jnp.float32),
            jax.ShapeDtypeStruct((n, 2), jnp.int32),
            jax.ShapeDtypeStruct((n, 2), jnp.float32),
            jax.ShapeDtypeStruct((1, _LANES), jnp.float32),
        ],
        interpret=interpret,
    )(x2, wt)

    return (
        gates2.reshape(b, s, 2),
        idx2.reshape(b, s, 2),
        load[0, :_NEXP],
        tl2.reshape(b, s, 2),
    )


# speedup vs baseline: 1.1460x; 1.0959x over previous
import functools

import jax
import jax.numpy as jnp
from jax.experimental import pallas as pl

_EMBED = 768
_NEXP = 8
_LANES = 128
_BLK = 4096


def _router_body(x_ref, wt_ref, g_ref, i_ref, tl_ref, load_ref):
    logits = jnp.dot(x_ref[:], wt_ref[:], preferred_element_type=jnp.float32)
    eidx = jax.lax.broadcasted_iota(jnp.int32, logits.shape, 1)
    neg = jnp.float32(-jnp.inf)
    masked = jnp.where(eidx < _NEXP, logits, neg)

    l1 = jnp.max(masked, axis=1, keepdims=True)
    i1 = jnp.min(jnp.where(masked == l1, eidx, _LANES), axis=1, keepdims=True)
    masked2 = jnp.where(eidx == i1, neg, masked)
    l2 = jnp.max(masked2, axis=1, keepdims=True)
    i2 = jnp.min(jnp.where(masked2 == l2, eidx, _LANES), axis=1, keepdims=True)

    e21 = jnp.exp(l2 - l1)
    denom = 1.0 + e21
    g1 = 1.0 / denom
    g2 = e21 / denom

    g_ref[:, 0:1] = g1
    g_ref[:, 1:2] = g2
    i_ref[:, 0:1] = i1
    i_ref[:, 1:2] = i2
    tl_ref[:, 0:1] = l1
    tl_ref[:, 1:2] = l2

    part = jnp.sum(
        jnp.where(eidx == i1, g1, 0.0) + jnp.where(eidx == i2, g2, 0.0),
        axis=0,
        keepdims=True,
    )

    @pl.when(pl.program_id(0) == 0)
    def _init():
        load_ref[:] = jnp.zeros_like(load_ref)

    load_ref[:] += part


@functools.partial(jax.jit, static_argnames=("interpret",))
def kernel(x, W, interpret=False):
    b, s, d = x.shape
    n = b * s
    x2 = x.reshape(n, d)
    wt = jnp.zeros((d, _LANES), jnp.float32).at[:, :_NEXP].set(W.T)

    grid = (n // _BLK,)
    gates2, idx2, tl2, load = pl.pallas_call(
        _router_body,
        grid=grid,
        in_specs=[
            pl.BlockSpec((_BLK, d), lambda i: (i, 0)),
            pl.BlockSpec((d, _LANES), lambda i: (0, 0)),
        ],
        out_specs=[
            pl.BlockSpec((_BLK, 2), lambda i: (i, 0)),
            pl.BlockSpec((_BLK, 2), lambda i: (i, 0)),
            pl.BlockSpec((_BLK, 2), lambda i: (i, 0)),
            pl.BlockSpec((1, _LANES), lambda i: (0, 0)),
        ],
        out_shape=[
            jax.ShapeDtypeStruct((n, 2), jnp.float32),
            jax.ShapeDtypeStruct((n, 2), jnp.int32),
            jax.ShapeDtypeStruct((n, 2), jnp.float32),
            jax.ShapeDtypeStruct((1, _LANES), jnp.float32),
        ],
        interpret=interpret,
    )(x2, wt)

    return (
        gates2.reshape(b, s, 2),
        idx2.reshape(b, s, 2),
        load[0, :_NEXP],
        tl2.reshape(b, s, 2),
    )


# speedup vs baseline: 2.8347x; 2.4735x over previous
import functools

import jax
import jax.numpy as jnp
from jax.experimental import pallas as pl

_EMBED = 768
_NEXP = 8
_BLK = 4096


def _router_body(x_ref, w_ref, g_ref, i_ref, tl_ref, load_ref):
    logits = jax.lax.dot_general(
        w_ref[:],
        x_ref[:],
        (((1,), (1,)), ((), ())),
        preferred_element_type=jnp.float32,
    )
    eidx = jax.lax.broadcasted_iota(jnp.int32, logits.shape, 0)
    neg = jnp.float32(-jnp.inf)

    l1 = jnp.max(logits, axis=0, keepdims=True)
    i1 = jnp.min(jnp.where(logits == l1, eidx, _NEXP), axis=0, keepdims=True)
    masked2 = jnp.where(eidx == i1, neg, logits)
    l2 = jnp.max(masked2, axis=0, keepdims=True)
    i2 = jnp.min(jnp.where(masked2 == l2, eidx, _NEXP), axis=0, keepdims=True)

    e21 = jnp.exp(l2 - l1)
    denom = 1.0 + e21
    g1 = 1.0 / denom
    g2 = e21 / denom

    g_ref[0:1, :] = g1
    g_ref[1:2, :] = g2
    i_ref[0:1, :] = i1
    i_ref[1:2, :] = i2
    tl_ref[0:1, :] = l1
    tl_ref[1:2, :] = l2

    part = jnp.sum(
        jnp.where(eidx == i1, g1, 0.0) + jnp.where(eidx == i2, g2, 0.0),
        axis=1,
        keepdims=True,
    )

    @pl.when(pl.program_id(0) == 0)
    def _init():
        load_ref[:] = jnp.zeros_like(load_ref)

    load_ref[:, 0:1] += part


@functools.partial(jax.jit, static_argnames=("interpret",))
def kernel(x, W, interpret=False):
    b, s, d = x.shape
    n = b * s
    x2 = x.reshape(n, d)

    grid = (n // _BLK,)
    g_t, i_t, tl_t, load = pl.pallas_call(
        _router_body,
        grid=grid,
        in_specs=[
            pl.BlockSpec((_BLK, d), lambda i: (i, 0)),
            pl.BlockSpec((_NEXP, d), lambda i: (0, 0)),
        ],
        out_specs=[
            pl.BlockSpec((2, _BLK), lambda i: (0, i)),
            pl.BlockSpec((2, _BLK), lambda i: (0, i)),
            pl.BlockSpec((2, _BLK), lambda i: (0, i)),
            pl.BlockSpec((_NEXP, 128), lambda i: (0, 0)),
        ],
        out_shape=[
            jax.ShapeDtypeStruct((2, n), jnp.float32),
            jax.ShapeDtypeStruct((2, n), jnp.int32),
            jax.ShapeDtypeStruct((2, n), jnp.float32),
            jax.ShapeDtypeStruct((_NEXP, 128), jnp.float32),
        ],
        interpret=interpret,
    )(x2, W)

    return (
        g_t.T.reshape(b, s, 2),
        i_t.T.reshape(b, s, 2),
        load[:, 0],
        tl_t.T.reshape(b, s, 2),
    )
